# baseline (speedup 1.0000x reference)
import functools

import jax
import jax.numpy as jnp
from jax import lax
from jax.experimental import pallas as pl
from jax.experimental.pallas import tpu as pltpu
from jax.experimental.pallas import tpu_sc as plsc

_NC = 2
_NS = 16
_NW = _NC * _NS
_NBG = 2
_NBT = 4
_NBI = 2
_L = 16
_CH = 128
_IB = 8


@functools.lru_cache(maxsize=None)
def _build(B, S, D):
    tok_per_w = (B // _NW) * S
    n_ch = tok_per_w // _CH
    n_ib = n_ch // _IB
    xrow_per_w = tok_per_w // 128

    mesh = plsc.VectorSubcoreMesh(core_axis_name="c", subcore_axis_name="s")

    def body(x2_hbm, table_hbm, pos_hbm, out_hbm,
             pos_v, idx_v, gat_v, tok_v, isem, gsem, osem):
        sid = lax.axis_index("s")
        wid = sid * _NC + lax.axis_index("c")
        base = wid * tok_per_w
        xrow0 = wid * xrow_per_w

        pltpu.sync_copy(pos_hbm, pos_v)

        def load_block(blk, bi):
            pltpu.async_copy(x2_hbm.at[pl.ds(xrow0 + blk * _IB, _IB)],
                             idx_v.at[bi], isem.at[bi])

        load_block(0, 0)

        def slot(s, carry):
            @pl.when(jnp.logical_and(s >= _NBT, s < n_ch))
            def _():
                bP = lax.rem(s, _NBT)
                pltpu.make_async_copy(
                    tok_v.at[bP], out_hbm.at[pl.ds(0, _CH)], osem.at[bP]
                ).wait()

            @pl.when(lax.rem(s, _IB) == 4)
            def _():
                blk = s // _IB + 1

                @pl.when(blk < n_ib)
                def _():
                    load_block(blk, lax.rem(blk, _NBI))

            @pl.when(jnp.logical_and(s >= 1, s <= n_ch))
            def _():
                c = s - 1
                j = lax.rem(c, _IB)
                bi = lax.rem(c // _IB, _NBI)
                bG = lax.rem(c, _NBG)

                @pl.when(j == 0)
                def _():
                    pltpu.make_async_copy(x2_hbm.at[pl.ds(0, _IB)],
                                          idx_v.at[bi], isem.at[bi]).wait()

                pltpu.async_copy(table_hbm.at[idx_v.at[bi, j]], gat_v.at[bG],
                                 gsem.at[bG])

            @pl.when(s >= 2)
            def _():
                o = s - 2
                bG = lax.rem(o, _NBG)
                bO = lax.rem(o, _NBT)
                pltpu.make_async_copy(table_hbm.at[idx_v.at[0, 0]],
                                      gat_v.at[bG], gsem.at[bG]).wait()
                r0 = lax.rem(o * _CH, S)
                n1 = jnp.minimum(S - r0, _CH)

                def add_row(pr_off):
                    def f(r, carry2):
                        for c in range(D // _L):
                            pv = pos_v[r + pr_off, pl.ds(c * _L, _L)]
                            tok_v[bO, r, pl.ds(c * _L, _L)] = (
                                gat_v[bG, r, pl.ds(c * _L, _L)] + pv)
                        return carry2
                    return f

                lax.fori_loop(0, n1, add_row(r0), 0)
                lax.fori_loop(n1, _CH, add_row(r0 - S), 0)

                off = base + o * _CH
                pltpu.async_copy(tok_v.at[bO], out_hbm.at[pl.ds(off, _CH)],
                                 osem.at[bO])

            return carry

        lax.fori_loop(0, n_ch + 2, slot, 0)

        for b in range(_NBT):
            pltpu.make_async_copy(
                tok_v.at[b], out_hbm.at[pl.ds(0, _CH)], osem.at[b]
            ).wait()

    return pl.kernel(
        body,
        out_type=jax.ShapeDtypeStruct((B * S, D), jnp.float32),
        mesh=mesh,
        scratch_types=[
            pltpu.VMEM((S, D), jnp.float32),
            pltpu.VMEM((_NBI, _IB, 128), jnp.int32),
            pltpu.VMEM((_NBG, _CH, 128), jnp.float32),
            pltpu.VMEM((_NBT, _CH, D), jnp.float32),
            pltpu.SemaphoreType.DMA((_NBI,)),
            pltpu.SemaphoreType.DMA((_NBG,)),
            pltpu.SemaphoreType.DMA((_NBT,)),
        ],
    )


def kernel(x, token_table, pos_table):
    B, S = x.shape
    V, D = token_table.shape
    x2 = x.reshape(B * S // 128, 128).astype(jnp.int32)
    table_p = jnp.pad(token_table, ((0, 0), (0, 128 - D)))
    out2 = _build(B, S, D)(x2, table_p, pos_table)
    return out2.reshape(B, S, D)
